# baseline (speedup 1.0000x reference)
import jax
import jax.numpy as jnp
from jax import lax
from jax.experimental import pallas as pl
from jax.experimental.pallas import tpu as pltpu
from jax.experimental.pallas import tpu_sc as plsc

_N_BINS = 15
_L = 16
_UNROLL = 7
_NCH = 4


def _ece_partials(conf, pred, lab, *, num_cores, num_subcores):
    nw = num_cores * num_subcores
    n = conf.shape[0]
    assert n % _L == 0
    total_vec = n // _L
    base_vec = total_vec // nw
    nbig = total_vec - base_vec * nw
    nv = base_vec + (1 if nbig else 0)
    short_elems = base_vec * _L

    cv = -(-nv // (_NCH * _UNROLL)) * _UNROLL
    cb = cv * _L
    last_small = short_elems - (_NCH - 1) * cb
    assert 0 < last_small <= cb and last_small % _L == 0
    zfill = (cb - last_small) // _L

    def body(conf_hbm, pred_hbm, lab_hbm, out_hbm,
             conf_v0, pred_v0, lab_v0, conf_v1, pred_v1, lab_v1,
             acc_v, buf_v, sem):
        slots = ((conf_v0, pred_v0, lab_v0), (conf_v1, pred_v1, lab_v1))
        wid = lax.axis_index("s") * num_cores + lax.axis_index("c")
        base = wid * short_elems + _L * jnp.minimum(wid, nbig)

        zero = jnp.zeros((_L,), jnp.float32)
        lane = lax.iota(jnp.int32, _L)
        tabv = lane.astype(jnp.float32) / jnp.float32(_N_BINS)

        def start_chunk(k):
            cv_, pv_, lv_ = slots[k % 2]
            st = base + k * cb
            if k < _NCH - 1:
                return [
                    pltpu.async_copy(conf_hbm.at[pl.ds(st, cb)], cv_, sem),
                    pltpu.async_copy(pred_hbm.at[pl.ds(st, cb)], pv_, sem),
                    pltpu.async_copy(lab_hbm.at[pl.ds(st, cb)], lv_, sem),
                ]
            for t in range(zfill):
                cv_[pl.ds(last_small + t * _L, _L)] = zero
            cps = [
                pltpu.async_copy(conf_hbm.at[pl.ds(st, last_small)],
                                 cv_.at[pl.ds(0, last_small)], sem),
                pltpu.async_copy(pred_hbm.at[pl.ds(st, last_small)],
                                 pv_.at[pl.ds(0, last_small)], sem),
                pltpu.async_copy(lab_hbm.at[pl.ds(st, last_small)],
                                 lv_.at[pl.ds(0, last_small)], sem),
            ]
            if nbig:
                @pl.when(wid < nbig)
                def _():
                    g = base + short_elems
                    o = last_small
                    pltpu.sync_copy(conf_hbm.at[pl.ds(g, _L)],
                                    cv_.at[pl.ds(o, _L)])
                    pltpu.sync_copy(pred_hbm.at[pl.ds(g, _L)],
                                    pv_.at[pl.ds(o, _L)])
                    pltpu.sync_copy(lab_hbm.at[pl.ds(g, _L)],
                                    lv_.at[pl.ds(o, _L)])
            return cps

        for r in range(_L):
            acc_v[r, :] = zero

        def one(slot, off):
            cv_, pv_, lv_ = slots[slot]
            c = cv_[pl.ds(off, _L)]
            p = pv_[pl.ds(off, _L)]
            l = lv_[pl.ds(off, _L)]
            a = jnp.where(p == l, jnp.float32(1.0), jnp.float32(0.0))
            d = c - a
            ti = (c * jnp.float32(15.0)).astype(jnp.int32)
            lo = jnp.take_along_axis(tabv, ti, axis=0)
            b = ti - (c == lo).astype(jnp.int32)
            plsc.addupdate_scatter(acc_v, [lane, b], d,
                                   mask=c > jnp.float32(0.0))

        cps = start_chunk(0)
        for k in range(_NCH):
            nxt = start_chunk(k + 1) if k + 1 < _NCH else None
            for cp in cps:
                cp.wait()
            slot = k % 2

            @plsc.parallel_loop(0, cb, _L, unroll=_UNROLL)
            def _(off):
                one(slot, off)

            cps = nxt

        tot = acc_v[0, :]
        for r in range(1, _L):
            tot = tot + acc_v[r, :]
        buf_v[...] = tot
        pltpu.sync_copy(buf_v, out_hbm.at[wid])

    mesh = plsc.VectorSubcoreMesh(
        core_axis_name="c", subcore_axis_name="s",
        num_cores=num_cores, num_subcores=num_subcores)
    kfn = pl.kernel(
        body,
        out_type=jax.ShapeDtypeStruct((nw, _L), jnp.float32),
        mesh=mesh,
        compiler_params=pltpu.CompilerParams(needs_layout_passes=False),
        scratch_types=[
            pltpu.VMEM((cb,), jnp.float32),
            pltpu.VMEM((cb,), jnp.int32),
            pltpu.VMEM((cb,), jnp.int32),
            pltpu.VMEM((cb,), jnp.float32),
            pltpu.VMEM((cb,), jnp.int32),
            pltpu.VMEM((cb,), jnp.int32),
            pltpu.VMEM((_L, _L), jnp.float32),
            pltpu.VMEM((_L,), jnp.float32),
            pltpu.SemaphoreType.DMA,
        ],
    )
    return kfn(conf, pred, lab)


@jax.jit
def kernel(confidences, predictions, labels):
    n = confidences.shape[0]
    parts = _ece_partials(confidences, predictions, labels,
                          num_cores=2, num_subcores=16)
    s = parts.sum(axis=0)
    ece = (jnp.abs(s[:_N_BINS - 1]).sum() + jnp.abs(s[14] + s[15])) \
        / jnp.float32(n)
    return ece.reshape(1)


# speedup vs baseline: 1.0231x; 1.0231x over previous
import jax
import jax.numpy as jnp
from jax import lax
from jax.experimental import pallas as pl
from jax.experimental.pallas import tpu as pltpu
from jax.experimental.pallas import tpu_sc as plsc

_N_BINS = 15
_L = 16
_UNROLL = 7
_NCH = 2


def _ece_partials(conf, pred, lab, *, num_cores, num_subcores):
    nw = num_cores * num_subcores
    n = conf.shape[0]
    assert n % _L == 0
    total_vec = n // _L
    base_vec = total_vec // nw
    nbig = total_vec - base_vec * nw
    nv = base_vec + (1 if nbig else 0)
    short_elems = base_vec * _L

    cv = -(-nv // (_NCH * _UNROLL)) * _UNROLL
    cb = cv * _L
    last_small = short_elems - (_NCH - 1) * cb
    assert 0 < last_small <= cb and last_small % _L == 0
    zfill = (cb - last_small) // _L

    def body(conf_hbm, pred_hbm, lab_hbm, out_hbm,
             conf_v0, pred_v0, lab_v0, conf_v1, pred_v1, lab_v1,
             acc_v, buf_v, sem):
        slots = ((conf_v0, pred_v0, lab_v0), (conf_v1, pred_v1, lab_v1))
        wid = lax.axis_index("s") * num_cores + lax.axis_index("c")
        base = wid * short_elems + _L * jnp.minimum(wid, nbig)

        zero = jnp.zeros((_L,), jnp.float32)
        lane = lax.iota(jnp.int32, _L)
        tabv = lane.astype(jnp.float32) / jnp.float32(_N_BINS)

        def start_chunk(k):
            cv_, pv_, lv_ = slots[k % 2]
            st = base + k * cb
            if k < _NCH - 1:
                return [
                    pltpu.async_copy(conf_hbm.at[pl.ds(st, cb)], cv_, sem),
                    pltpu.async_copy(pred_hbm.at[pl.ds(st, cb)], pv_, sem),
                    pltpu.async_copy(lab_hbm.at[pl.ds(st, cb)], lv_, sem),
                ]
            for t in range(zfill):
                cv_[pl.ds(last_small + t * _L, _L)] = zero
            cps = [
                pltpu.async_copy(conf_hbm.at[pl.ds(st, last_small)],
                                 cv_.at[pl.ds(0, last_small)], sem),
                pltpu.async_copy(pred_hbm.at[pl.ds(st, last_small)],
                                 pv_.at[pl.ds(0, last_small)], sem),
                pltpu.async_copy(lab_hbm.at[pl.ds(st, last_small)],
                                 lv_.at[pl.ds(0, last_small)], sem),
            ]
            if nbig:
                @pl.when(wid < nbig)
                def _():
                    g = base + short_elems
                    o = last_small
                    pltpu.sync_copy(conf_hbm.at[pl.ds(g, _L)],
                                    cv_.at[pl.ds(o, _L)])
                    pltpu.sync_copy(pred_hbm.at[pl.ds(g, _L)],
                                    pv_.at[pl.ds(o, _L)])
                    pltpu.sync_copy(lab_hbm.at[pl.ds(g, _L)],
                                    lv_.at[pl.ds(o, _L)])
            return cps

        for r in range(_L):
            acc_v[r, :] = zero

        def one(slot, off):
            cv_, pv_, lv_ = slots[slot]
            c = cv_[pl.ds(off, _L)]
            p = pv_[pl.ds(off, _L)]
            l = lv_[pl.ds(off, _L)]
            a = jnp.where(p == l, jnp.float32(1.0), jnp.float32(0.0))
            d = c - a
            ti = (c * jnp.float32(15.0)).astype(jnp.int32)
            lo = jnp.take_along_axis(tabv, ti, axis=0)
            b = ti - (c == lo).astype(jnp.int32)
            plsc.addupdate_scatter(acc_v, [lane, b], d,
                                   mask=c > jnp.float32(0.0))

        cps = start_chunk(0)
        for k in range(_NCH):
            nxt = start_chunk(k + 1) if k + 1 < _NCH else None
            for cp in cps:
                cp.wait()
            slot = k % 2

            @plsc.parallel_loop(0, cb, _L, unroll=_UNROLL)
            def _(off):
                one(slot, off)

            cps = nxt

        tot = acc_v[0, :]
        for r in range(1, _L):
            tot = tot + acc_v[r, :]
        buf_v[...] = tot
        pltpu.sync_copy(buf_v, out_hbm.at[wid])

    mesh = plsc.VectorSubcoreMesh(
        core_axis_name="c", subcore_axis_name="s",
        num_cores=num_cores, num_subcores=num_subcores)
    kfn = pl.kernel(
        body,
        out_type=jax.ShapeDtypeStruct((nw, _L), jnp.float32),
        mesh=mesh,
        compiler_params=pltpu.CompilerParams(needs_layout_passes=False),
        scratch_types=[
            pltpu.VMEM((cb,), jnp.float32),
            pltpu.VMEM((cb,), jnp.int32),
            pltpu.VMEM((cb,), jnp.int32),
            pltpu.VMEM((cb,), jnp.float32),
            pltpu.VMEM((cb,), jnp.int32),
            pltpu.VMEM((cb,), jnp.int32),
            pltpu.VMEM((_L, _L), jnp.float32),
            pltpu.VMEM((_L,), jnp.float32),
            pltpu.SemaphoreType.DMA,
        ],
    )
    return kfn(conf, pred, lab)


@jax.jit
def kernel(confidences, predictions, labels):
    n = confidences.shape[0]
    parts = _ece_partials(confidences, predictions, labels,
                          num_cores=2, num_subcores=16)
    s = parts.sum(axis=0)
    ece = (jnp.abs(s[:_N_BINS - 1]).sum() + jnp.abs(s[14] + s[15])) \
        / jnp.float32(n)
    return ece.reshape(1)
